# baseline (speedup 1.0000x reference)
import functools

import jax
import jax.numpy as jnp
from jax import lax
from jax.experimental import pallas as pl
from jax.experimental.pallas import tpu as pltpu
from jax.experimental.pallas import tpu_sc as plsc

L = 16


@functools.cache
def _build(batch, z_dim):
    info = plsc.get_sparse_core_info()
    NC, NS = info.num_cores, info.num_subcores
    NW = NC * NS
    rows_per_w = batch // NW
    R = 8
    nblocks = rows_per_w // R
    nchunks = z_dim // L
    BW = R * z_dim

    mesh = plsc.VectorSubcoreMesh(core_axis_name="c", subcore_axis_name="s")

    @functools.partial(
        pl.kernel,
        mesh=mesh,
        compiler_params=pltpu.CompilerParams(needs_layout_passes=False),
        out_type=jax.ShapeDtypeStruct((batch * z_dim,), jnp.float32),
        scratch_types=[
            pltpu.VMEM((z_dim,), jnp.int32),
            pltpu.VMEM((BW,), jnp.float32),
            pltpu.VMEM((BW,), jnp.float32),
            pltpu.VMEM((BW,), jnp.float32),
            pltpu.VMEM((BW,), jnp.float32),
            pltpu.SemaphoreType.DMA,
            pltpu.SemaphoreType.DMA,
            pltpu.SemaphoreType.DMA,
            pltpu.SemaphoreType.DMA,
        ],
    )
    def k(z_hbm, perm_hbm, out_hbm, perm_v, in0, in1, o0, o1, is0, is1, os0, os1):
        ins, outs = (in0, in1), (o0, o1)
        isems, osems = (is0, is1), (os0, os1)
        wid = lax.axis_index("s") * NC + lax.axis_index("c")
        base = wid * (rows_per_w * z_dim)

        pltpu.sync_copy(perm_hbm, perm_v)

        def start_in(b, q):
            pltpu.async_copy(z_hbm.at[pl.ds(base + b * BW, BW)], ins[q], isems[q])

        def wait_in(b, q):
            pltpu.make_async_copy(
                z_hbm.at[pl.ds(base + b * BW, BW)], ins[q], isems[q]
            ).wait()

        def start_out(b, q):
            pltpu.async_copy(outs[q], out_hbm.at[pl.ds(base + b * BW, BW)], osems[q])

        def wait_out(b, q):
            pltpu.make_async_copy(
                outs[q], out_hbm.at[pl.ds(base + b * BW, BW)], osems[q]
            ).wait()

        def compute(q):
            in_v, out_v = ins[q], outs[q]

            def chunk(c, _):
                idx = perm_v[pl.ds(c * L, L)]
                for r in range(R):
                    out_v[pl.ds(c * L + r * z_dim, L)] = plsc.load_gather(
                        in_v, [idx + r * z_dim]
                    )
                return _

            lax.fori_loop(0, nchunks, chunk, None, unroll=2)

        start_in(0, 0)
        start_in(1, 1)

        @pl.loop(0, nblocks, step=2)
        def body(g):
            for q in range(2):
                b = g + q
                wait_in(b, q)

                @pl.when(b >= 2)
                def _():
                    wait_out(b - 2, q)

                compute(q)
                start_out(b, q)

                @pl.when(b + 2 < nblocks)
                def _():
                    start_in(b + 2, q)

        wait_out(nblocks - 2, 0)
        wait_out(nblocks - 1, 1)

    return k


def kernel(z, perm):
    batch, z_dim = z.shape
    k = _build(batch, z_dim)
    out_flat = k(z.reshape(-1), perm.astype(jnp.int32))
    return out_flat.reshape(batch, z_dim)


# speedup vs baseline: 1.6443x; 1.6443x over previous
import functools

import jax
import jax.numpy as jnp
from jax import lax
from jax.experimental import pallas as pl
from jax.experimental.pallas import tpu as pltpu
from jax.experimental.pallas import tpu_sc as plsc

L = 16


@functools.cache
def _build(batch, z_dim):
    info = plsc.get_sparse_core_info()
    NC, NS = info.num_cores, info.num_subcores
    NW = NC * NS
    rows_per_w = batch // NW
    R = 8
    nblocks = rows_per_w // R
    nchunks = z_dim // L
    BW = R * z_dim

    mesh = plsc.VectorSubcoreMesh(core_axis_name="c", subcore_axis_name="s")

    @functools.partial(
        pl.kernel,
        mesh=mesh,
        compiler_params=pltpu.CompilerParams(needs_layout_passes=False),
        out_type=jax.ShapeDtypeStruct((batch * z_dim,), jnp.float32),
        scratch_types=[
            pltpu.VMEM((z_dim,), jnp.int32),
            pltpu.VMEM((BW,), jnp.float32),
            pltpu.VMEM((BW,), jnp.float32),
            pltpu.VMEM((BW,), jnp.float32),
            pltpu.VMEM((BW,), jnp.float32),
            pltpu.SemaphoreType.DMA,
            pltpu.SemaphoreType.DMA,
            pltpu.SemaphoreType.DMA,
            pltpu.SemaphoreType.DMA,
        ],
    )
    def k(z_hbm, perm_hbm, out_hbm, perm_v, in0, in1, o0, o1, is0, is1, os0, os1):
        ins, outs = (in0, in1), (o0, o1)
        isems, osems = (is0, is1), (os0, os1)
        wid = lax.axis_index("s") * NC + lax.axis_index("c")
        base = wid * (rows_per_w * z_dim)

        pltpu.sync_copy(perm_hbm, perm_v)

        def start_in(b, q):
            pltpu.async_copy(z_hbm.at[pl.ds(base + b * BW, BW)], ins[q], isems[q])

        def wait_in(b, q):
            pltpu.make_async_copy(
                z_hbm.at[pl.ds(base + b * BW, BW)], ins[q], isems[q]
            ).wait()

        def start_out(b, q):
            pltpu.async_copy(outs[q], out_hbm.at[pl.ds(base + b * BW, BW)], osems[q])

        def wait_out(b, q):
            pltpu.make_async_copy(
                outs[q], out_hbm.at[pl.ds(base + b * BW, BW)], osems[q]
            ).wait()

        def compute(q):
            in_v, out_v = ins[q], outs[q]

            @plsc.parallel_loop(0, nchunks, unroll=4)
            def chunk(c):
                idx = perm_v[pl.ds(c * L, L)]
                for r in range(R):
                    out_v[pl.ds(c * L + r * z_dim, L)] = plsc.load_gather(
                        in_v, [idx + r * z_dim]
                    )

        start_in(0, 0)
        start_in(1, 1)

        @pl.loop(0, nblocks, step=2)
        def body(g):
            for q in range(2):
                b = g + q
                wait_in(b, q)

                @pl.when(b >= 2)
                def _():
                    wait_out(b - 2, q)

                compute(q)
                start_out(b, q)

                @pl.when(b + 2 < nblocks)
                def _():
                    start_in(b + 2, q)

        wait_out(nblocks - 2, 0)
        wait_out(nblocks - 1, 1)

    return k


def kernel(z, perm):
    batch, z_dim = z.shape
    k = _build(batch, z_dim)
    out_flat = k(z.reshape(-1), perm.astype(jnp.int32))
    return out_flat.reshape(batch, z_dim)


# speedup vs baseline: 4.7984x; 2.9181x over previous
import functools

import jax
import jax.numpy as jnp
from jax import lax
from jax.experimental import pallas as pl
from jax.experimental.pallas import tpu as pltpu
from jax.experimental.pallas import tpu_sc as plsc

L = 16


@functools.cache
def _build(batch, z_dim):
    info = plsc.get_sparse_core_info()
    NC, NS = info.num_cores, info.num_subcores
    NW = NC * NS
    rows_per_w = batch // NW
    R = 8
    nblocks = rows_per_w // R
    nchunks = z_dim // L

    mesh = plsc.VectorSubcoreMesh(core_axis_name="c", subcore_axis_name="s")

    @functools.partial(
        pl.kernel,
        mesh=mesh,
        compiler_params=pltpu.CompilerParams(
            needs_layout_passes=False,
            use_tc_tiling_on_sc=True,
        ),
        out_type=jax.ShapeDtypeStruct((batch, z_dim), jnp.float32),
        scratch_types=[
            pltpu.VMEM((z_dim,), jnp.int32),
            pltpu.VMEM((R, z_dim), jnp.float32),
            pltpu.VMEM((R, z_dim), jnp.float32),
            pltpu.VMEM((R, z_dim), jnp.float32),
            pltpu.VMEM((R, z_dim), jnp.float32),
            pltpu.SemaphoreType.DMA,
            pltpu.SemaphoreType.DMA,
            pltpu.SemaphoreType.DMA,
            pltpu.SemaphoreType.DMA,
        ],
    )
    def k(z_hbm, perm_hbm, out_hbm, perm_v, in0, in1, o0, o1, is0, is1, os0, os1):
        ins, outs = (in0, in1), (o0, o1)
        isems, osems = (is0, is1), (os0, os1)
        wid = lax.axis_index("s") * NC + lax.axis_index("c")
        base = wid * rows_per_w

        pltpu.sync_copy(perm_hbm, perm_v)

        def start_in(b, q):
            pltpu.async_copy(z_hbm.at[pl.ds(base + b * R, R)], ins[q], isems[q])

        def wait_in(b, q):
            pltpu.make_async_copy(
                z_hbm.at[pl.ds(base + b * R, R)], ins[q], isems[q]
            ).wait()

        def start_out(b, q):
            pltpu.async_copy(outs[q], out_hbm.at[pl.ds(base + b * R, R)], osems[q])

        def wait_out(b, q):
            pltpu.make_async_copy(
                outs[q], out_hbm.at[pl.ds(base + b * R, R)], osems[q]
            ).wait()

        def compute(q):
            in_v, out_v = ins[q], outs[q]

            @plsc.parallel_loop(0, nchunks, unroll=4)
            def chunk(c):
                idx = perm_v[pl.ds(c * L, L)]
                for r in range(R):
                    row = jnp.full((L,), r, jnp.int32)
                    out_v[r, pl.ds(c * L, L)] = plsc.load_gather(in_v, [row, idx])

        start_in(0, 0)
        start_in(1, 1)

        @pl.loop(0, nblocks, step=2)
        def body(g):
            for q in range(2):
                b = g + q
                wait_in(b, q)

                @pl.when(b >= 2)
                def _():
                    wait_out(b - 2, q)

                compute(q)
                start_out(b, q)

                @pl.when(b + 2 < nblocks)
                def _():
                    start_in(b + 2, q)

        wait_out(nblocks - 2, 0)
        wait_out(nblocks - 1, 1)

    return k


def kernel(z, perm):
    batch, z_dim = z.shape
    k = _build(batch, z_dim)
    return k(z, perm.astype(jnp.int32))
